# baseline (speedup 1.0000x reference)
import functools

import jax
import jax.numpy as jnp
from jax import lax
from jax.experimental import pallas as pl
from jax.experimental.pallas import tpu as pltpu
from jax.experimental.pallas import tpu_sc as plsc

H = 1024
I = 512
E = 8
NS = 1

BLK = 512
T = 256
CH = 256
NW = 32
CHW = 32


def _router_kernel(x_ref, wr_ref, dep_ref, pos_ref, w01_ref, be_ref, *,
                   n_tok, n_asn, n_blocks):
    x32 = x_ref[...]

    logits = lax.dot_general(x32, wr_ref[...], (((1,), (1,)), ((), ())),
                             preferred_element_type=jnp.float32)
    logits = logits + jnp.max(dep_ref[0, 0:1, :].astype(jnp.float32)) * 0.0
    m = jnp.max(logits, axis=1, keepdims=True)
    p = jnp.exp(logits - m)
    p = p / jnp.sum(p, axis=1, keepdims=True)
    eids = lax.broadcasted_iota(jnp.int32, (n_tok, E), 1)
    w0 = jnp.max(p, axis=1, keepdims=True)
    i0 = jnp.argmax(p, axis=1, keepdims=True)
    p2 = jnp.where(eids == i0, -1.0, p)
    w1 = jnp.max(p2, axis=1, keepdims=True)
    i1 = jnp.argmax(p2, axis=1, keepdims=True)
    w01_ref[...] = jnp.concatenate([w0, w1], axis=1)

    onehot0 = (eids == i0).astype(jnp.float32)
    onehot1 = (eids == i1).astype(jnp.float32)

    r_io = lax.broadcasted_iota(jnp.int32, (CH, CH), 0)
    c_io = lax.broadcasted_iota(jnp.int32, (CH, CH), 1)
    ltri = (r_io > c_io).astype(jnp.float32)

    n_chunks = n_asn // CH
    per_k = n_tok // CH
    tot = jnp.zeros((1, E), jnp.float32)

    def chunk(c):
        if c < per_k:
            return lax.slice(onehot0, (c * CH, 0), ((c + 1) * CH, E))
        return lax.slice(onehot1, ((c - per_k) * CH, 0), ((c - per_k + 1) * CH, E))

    rk = CH // 128
    rsel = []
    for c in range(n_chunks):
        oc = chunk(c)
        ranks = lax.dot_general(ltri, oc, (((1,), (0,)), ((), ())),
                                preferred_element_type=jnp.float32) + tot
        rsel.append(jnp.sum(oc * ranks, axis=1, keepdims=True))
        tot = tot + jnp.sum(oc, axis=0, keepdims=True)

    cnt = tot.astype(jnp.int32)
    nb = lax.shift_right_logical(cnt + (T - 1), T.bit_length() - 1)
    e_r = lax.broadcasted_iota(jnp.int32, (E, E), 0)
    e_c = lax.broadcasted_iota(jnp.int32, (E, E), 1)
    utri = (e_r < e_c).astype(jnp.float32)
    bstart = lax.dot_general(nb.astype(jnp.float32), utri,
                             (((1,), (0,)), ((), ())),
                             preferred_element_type=jnp.float32)
    row_start_f = bstart * float(T)

    for c in range(n_chunks):
        oc = chunk(c)
        posc = rsel[c] + jnp.sum(oc * row_start_f, axis=1, keepdims=True)
        pos_ref[pl.ds(c * rk, rk), :] = (
            posc.astype(jnp.int32).reshape(rk, 128))

    nbe = n_blocks + E
    b_io = lax.broadcasted_iota(jnp.int32, (nbe, E), 0)
    bstart_i = bstart.astype(jnp.int32)
    be = jnp.sum((bstart_i <= b_io).astype(jnp.int32),
                 axis=1, keepdims=True) - 1
    bt = jnp.sum(nb, axis=1, keepdims=True)
    row_io = lax.broadcasted_iota(jnp.int32, (nbe, 1), 0)
    be_ref[...] = jnp.where(row_io == nbe - 1, bt, be)


def _shared_kernel(x_ref, wsg_ref, wsu_ref, wsd_ref, out_ref):
    x32 = x_ref[...]
    xb = x32.astype(jnp.bfloat16)
    acc = x32
    for s in range(NS):
        g = lax.dot_general(xb, wsg_ref[s], (((1,), (1,)), ((), ())),
                            preferred_element_type=jnp.float32)
        u = lax.dot_general(xb, wsu_ref[s], (((1,), (1,)), ((), ())),
                            preferred_element_type=jnp.float32)
        h = (jax.nn.sigmoid(g) * u).astype(jnp.bfloat16)
        acc = acc + lax.dot_general(h, wsd_ref[s], (((1,), (1,)), ((), ())),
                                    preferred_element_type=jnp.float32)
    out_ref[...] = acc


def _sc_scatter(xf, pos, n_rows_out):
    n_asn = pos.shape[0]
    n_tok, d = xf.shape
    per_w = n_tok // NW
    mesh = plsc.VectorSubcoreMesh(core_axis_name="c", subcore_axis_name="s")

    @functools.partial(
        pl.kernel, mesh=mesh,
        out_type=jax.ShapeDtypeStruct((n_rows_out, d), jnp.float32),
        scratch_types=[pltpu.VMEM((2, per_w), jnp.int32),
                       pltpu.VMEM((per_w, d), jnp.float32),
                       pltpu.SemaphoreType.DMA,
                       pltpu.SemaphoreType.DMA,
                       pltpu.SemaphoreType.DMA],
    )
    def k(x_hbm, pos_hbm, out_hbm, idx_v, rows_v, s_in, s0, s1):
        wid = lax.axis_index("s") * 2 + lax.axis_index("c")
        tb = wid * per_w
        ci0 = pltpu.async_copy(pos_hbm.at[pl.ds(tb, per_w)], idx_v.at[0], s0)
        ci1 = pltpu.async_copy(pos_hbm.at[pl.ds(n_tok + tb, per_w)],
                               idx_v.at[1], s1)
        cr = pltpu.async_copy(x_hbm.at[pl.ds(tb, per_w)], rows_v, s_in)
        ci0.wait()
        ci1.wait()
        cr.wait()
        c0 = pltpu.async_copy(rows_v, out_hbm.at[idx_v.at[0]], s0)
        c1 = pltpu.async_copy(rows_v, out_hbm.at[idx_v.at[1]], s1)
        c0.wait()
        c1.wait()

    return k(xf, pos)


def _sc_gather(ys, pos):
    n_asn = pos.shape[0]
    d = ys.shape[1]
    per_w = n_asn // NW
    n_ch = per_w // CHW
    mesh = plsc.VectorSubcoreMesh(core_axis_name="c", subcore_axis_name="s")

    @functools.partial(
        pl.kernel, mesh=mesh,
        out_type=jax.ShapeDtypeStruct((n_asn, d), jnp.float32),
        scratch_types=[pltpu.VMEM((n_ch, CHW), jnp.int32),
                       pltpu.VMEM((CHW, d), jnp.float32),
                       pltpu.VMEM((CHW, d), jnp.float32),
                       pltpu.SemaphoreType.DMA,
                       pltpu.SemaphoreType.DMA],
    )
    def k(ys_hbm, pos_hbm, out_hbm, idx_v, buf_a, buf_b, sem_a, sem_b):
        wid = lax.axis_index("s") * 2 + lax.axis_index("c")
        base = wid * per_w
        for c in range(n_ch):
            pltpu.sync_copy(pos_hbm.at[pl.ds(base + c * CHW, CHW)],
                            idx_v.at[c])
        bufs = (buf_a, buf_b)
        sems = (sem_a, sem_b)
        cps = [pltpu.async_copy(ys_hbm.at[idx_v.at[0]], buf_a, sem_a)]
        for c in range(n_ch):
            cps[c].wait()
            if c + 1 < n_ch:
                cps.append(pltpu.async_copy(ys_hbm.at[idx_v.at[c + 1]],
                                            bufs[(c + 1) % 2],
                                            sems[(c + 1) % 2]))
            pltpu.sync_copy(bufs[c % 2],
                            out_hbm.at[pl.ds(base + c * CHW, CHW)])

    return k(ys, pos)


def _cast_kernel(wg_ref, wu_ref, wd_ref, og_ref, ou_ref, od_ref):
    og_ref[...] = wg_ref[...].astype(jnp.bfloat16)
    ou_ref[...] = wu_ref[...].astype(jnp.bfloat16)
    od_ref[...] = wd_ref[...].astype(jnp.bfloat16)


def _cast_weights(w_gate, w_up, w_down):
    return pl.pallas_call(
        _cast_kernel,
        grid=(E,),
        in_specs=[pl.BlockSpec((1, I, H), lambda e: (e, 0, 0)),
                  pl.BlockSpec((1, I, H), lambda e: (e, 0, 0)),
                  pl.BlockSpec((1, H, I), lambda e: (e, 0, 0))],
        out_specs=[pl.BlockSpec((1, I, H), lambda e: (e, 0, 0)),
                   pl.BlockSpec((1, I, H), lambda e: (e, 0, 0)),
                   pl.BlockSpec((1, H, I), lambda e: (e, 0, 0))],
        out_shape=[jax.ShapeDtypeStruct((E, I, H), jnp.bfloat16),
                   jax.ShapeDtypeStruct((E, I, H), jnp.bfloat16),
                   jax.ShapeDtypeStruct((E, H, I), jnp.bfloat16)],
    )(w_gate, w_up, w_down)


def _group_mm_kernel(be_ref, xs_ref, wg_ref, wu_ref, wd_ref, out_ref, *, nbe):
    b = pl.program_id(0)
    bt = be_ref[nbe - 1]

    @pl.when(b < bt)
    def _():
        e = be_ref[b]
        xb = xs_ref[...].astype(jnp.bfloat16)
        g = lax.dot_general(xb, wg_ref[e], (((1,), (1,)), ((), ())),
                            preferred_element_type=jnp.float32)
        u = lax.dot_general(xb, wu_ref[e], (((1,), (1,)), ((), ())),
                            preferred_element_type=jnp.float32)
        h = (jax.nn.sigmoid(g) * u).astype(jnp.bfloat16)
        de = lax.dot_general(h, wd_ref[e], (((1,), (1,)), ((), ())),
                             preferred_element_type=jnp.float32)
        out_ref[...] = de.astype(jnp.bfloat16).astype(jnp.float32)


def _combine_kernel(acc_ref, y0_ref, y1_ref, w01_ref, out_ref):
    w01 = w01_ref[...].astype(jnp.bfloat16)
    c0 = (w01[:, 0:1] * y0_ref[...].astype(jnp.bfloat16)).astype(jnp.float32)
    c1 = (w01[:, 1:2] * y1_ref[...].astype(jnp.bfloat16)).astype(jnp.float32)
    out_ref[...] = acc_ref[...] + c0 + c1


@jax.jit
def kernel(x, ws_gate, ws_up, ws_down, w_router, w_gate, w_up, w_down):
    B, S, Hx = x.shape
    n_tok = B * S
    n_asn = 2 * n_tok
    n_blocks = n_asn // T + E
    flat = x.reshape(n_tok, Hx)
    wsg = ws_gate.astype(jnp.bfloat16)
    wsu = ws_up.astype(jnp.bfloat16)
    wsd = ws_down.astype(jnp.bfloat16)
    wg, wu, wd = _cast_weights(w_gate, w_up, w_down)

    pos, w01, be = pl.pallas_call(
        functools.partial(_router_kernel, n_tok=n_tok, n_asn=n_asn,
                          n_blocks=n_blocks),
        grid=(1,),
        in_specs=[pl.BlockSpec((n_tok, H), lambda i: (0, 0)),
                  pl.BlockSpec((E, H), lambda i: (0, 0)),
                  pl.BlockSpec((1, 16, 128), lambda i: (0, 0, 0))],
        out_specs=[pl.BlockSpec((n_asn // 128, 128), lambda i: (0, 0)),
                   pl.BlockSpec((n_tok, 2), lambda i: (0, 0)),
                   pl.BlockSpec((n_blocks + E, 1), lambda i: (0, 0))],
        out_shape=[jax.ShapeDtypeStruct((n_asn // 128, 128), jnp.int32),
                   jax.ShapeDtypeStruct((n_tok, 2), jnp.float32),
                   jax.ShapeDtypeStruct((n_blocks + E, 1), jnp.int32)],
    )(flat, w_router, wg)

    nblk = n_tok // BLK
    full = lambda shape: pl.BlockSpec(shape, lambda i: (0,) * len(shape))
    acc2 = pl.pallas_call(
        _shared_kernel,
        grid=(nblk,),
        in_specs=[pl.BlockSpec((BLK, H), lambda i: (i, 0)),
                  full((NS, I, H)), full((NS, I, H)), full((NS, H, I))],
        out_specs=pl.BlockSpec((BLK, H), lambda i: (i, 0)),
        out_shape=jax.ShapeDtypeStruct((n_tok, H), jnp.float32),
    )(flat, wsg, wsu, wsd)

    pos_flat = pos.reshape(n_asn)
    xs = _sc_scatter(flat, pos_flat, n_blocks * T)

    nbe = n_blocks + E

    def _beff(b, be_r):
        return jnp.minimum(b, jnp.maximum(be_r[nbe - 1] - 1, 0))

    ys = pl.pallas_call(
        functools.partial(_group_mm_kernel, nbe=nbe),
        grid_spec=pltpu.PrefetchScalarGridSpec(
            num_scalar_prefetch=1,
            grid=(n_blocks,),
            in_specs=[
                pl.BlockSpec((T, H), lambda b, be_r: (_beff(b, be_r), 0)),
                pl.BlockSpec((E, I, H), lambda b, be_r: (0, 0, 0)),
                pl.BlockSpec((E, I, H), lambda b, be_r: (0, 0, 0)),
                pl.BlockSpec((E, H, I), lambda b, be_r: (0, 0, 0)),
            ],
            out_specs=pl.BlockSpec((T, H), lambda b, be_r: (_beff(b, be_r), 0)),
        ),
        out_shape=jax.ShapeDtypeStruct((n_blocks * T, H), jnp.float32),
    )(be.reshape(nbe), xs, wg, wu, wd)

    y01 = _sc_gather(ys, pos_flat)

    kblk = n_tok // BLK
    out = pl.pallas_call(
        _combine_kernel,
        grid=(nblk,),
        in_specs=[pl.BlockSpec((BLK, H), lambda i: (i, 0)),
                  pl.BlockSpec((BLK, H), lambda i: (i, 0)),
                  pl.BlockSpec((BLK, H), lambda i: (i + kblk, 0)),
                  pl.BlockSpec((BLK, 2), lambda i: (i, 0))],
        out_specs=pl.BlockSpec((BLK, H), lambda i: (i, 0)),
        out_shape=jax.ShapeDtypeStruct((n_tok, H), jnp.float32),
    )(acc2, y01, y01, w01)
    return out.reshape(B, S, Hx)


# speedup vs baseline: 1.0452x; 1.0452x over previous
import functools

import jax
import jax.numpy as jnp
from jax import lax
from jax.experimental import pallas as pl
from jax.experimental.pallas import tpu as pltpu
from jax.experimental.pallas import tpu_sc as plsc

H = 1024
I = 512
E = 8
NS = 1

BLK = 512
T = 512
CH = 256
NW = 32
CHW = 32


def _router_kernel(x_ref, wr_ref, pos_ref, w01_ref, be_ref, *,
                   n_tok, n_asn, n_blocks):
    x32 = x_ref[...]

    logits = lax.dot_general(x32, wr_ref[...], (((1,), (1,)), ((), ())),
                             preferred_element_type=jnp.float32)
    m = jnp.max(logits, axis=1, keepdims=True)
    p = jnp.exp(logits - m)
    p = p / jnp.sum(p, axis=1, keepdims=True)
    eids = lax.broadcasted_iota(jnp.int32, (n_tok, E), 1)
    w0 = jnp.max(p, axis=1, keepdims=True)
    i0 = jnp.argmax(p, axis=1, keepdims=True)
    p2 = jnp.where(eids == i0, -1.0, p)
    w1 = jnp.max(p2, axis=1, keepdims=True)
    i1 = jnp.argmax(p2, axis=1, keepdims=True)
    w01_ref[...] = jnp.concatenate([w0, w1], axis=1)

    onehot0 = (eids == i0).astype(jnp.float32)
    onehot1 = (eids == i1).astype(jnp.float32)

    r_io = lax.broadcasted_iota(jnp.int32, (CH, CH), 0)
    c_io = lax.broadcasted_iota(jnp.int32, (CH, CH), 1)
    ltri = (r_io > c_io).astype(jnp.float32)

    n_chunks = n_asn // CH
    per_k = n_tok // CH
    tot = jnp.zeros((1, E), jnp.float32)

    def chunk(c):
        if c < per_k:
            return lax.slice(onehot0, (c * CH, 0), ((c + 1) * CH, E))
        return lax.slice(onehot1, ((c - per_k) * CH, 0), ((c - per_k + 1) * CH, E))

    rk = CH // 128
    rsel = []
    for c in range(n_chunks):
        oc = chunk(c)
        ranks = lax.dot_general(ltri, oc, (((1,), (0,)), ((), ())),
                                preferred_element_type=jnp.float32) + tot
        rsel.append(jnp.sum(oc * ranks, axis=1, keepdims=True))
        tot = tot + jnp.sum(oc, axis=0, keepdims=True)

    cnt = tot.astype(jnp.int32)
    nb = lax.shift_right_logical(cnt + (T - 1), T.bit_length() - 1)
    e_r = lax.broadcasted_iota(jnp.int32, (E, E), 0)
    e_c = lax.broadcasted_iota(jnp.int32, (E, E), 1)
    utri = (e_r < e_c).astype(jnp.float32)
    bstart = lax.dot_general(nb.astype(jnp.float32), utri,
                             (((1,), (0,)), ((), ())),
                             preferred_element_type=jnp.float32)
    row_start_f = bstart * float(T)

    for c in range(n_chunks):
        oc = chunk(c)
        posc = rsel[c] + jnp.sum(oc * row_start_f, axis=1, keepdims=True)
        pos_ref[pl.ds(c * rk, rk), :] = (
            posc.astype(jnp.int32).reshape(rk, 128))

    nbe = n_blocks + E
    b_io = lax.broadcasted_iota(jnp.int32, (nbe, E), 0)
    bstart_i = bstart.astype(jnp.int32)
    be = jnp.sum((bstart_i <= b_io).astype(jnp.int32),
                 axis=1, keepdims=True) - 1
    bt = jnp.sum(nb, axis=1, keepdims=True)
    row_io = lax.broadcasted_iota(jnp.int32, (nbe, 1), 0)
    be_ref[...] = jnp.where(row_io == nbe - 1, bt, be)


def _shared_kernel(x_ref, wsg_ref, wsu_ref, wsd_ref, out_ref):
    x32 = x_ref[...]
    xb = x32.astype(jnp.bfloat16)
    acc = x32
    for s in range(NS):
        g = lax.dot_general(xb, wsg_ref[s], (((1,), (1,)), ((), ())),
                            preferred_element_type=jnp.float32)
        u = lax.dot_general(xb, wsu_ref[s], (((1,), (1,)), ((), ())),
                            preferred_element_type=jnp.float32)
        h = (jax.nn.sigmoid(g) * u).astype(jnp.bfloat16)
        acc = acc + lax.dot_general(h, wsd_ref[s], (((1,), (1,)), ((), ())),
                                    preferred_element_type=jnp.float32)
    out_ref[...] = acc


def _sc_scatter(xf, pos, n_rows_out):
    n_asn = pos.shape[0]
    n_tok, d = xf.shape
    per_w = n_tok // NW
    mesh = plsc.VectorSubcoreMesh(core_axis_name="c", subcore_axis_name="s")

    @functools.partial(
        pl.kernel, mesh=mesh,
        out_type=jax.ShapeDtypeStruct((n_rows_out, d), jnp.float32),
        scratch_types=[pltpu.VMEM((2, per_w), jnp.int32),
                       pltpu.VMEM((per_w, d), jnp.float32),
                       pltpu.SemaphoreType.DMA,
                       pltpu.SemaphoreType.DMA,
                       pltpu.SemaphoreType.DMA],
    )
    def k(x_hbm, pos_hbm, out_hbm, idx_v, rows_v, s_in, s0, s1):
        wid = lax.axis_index("s") * 2 + lax.axis_index("c")
        tb = wid * per_w
        ci0 = pltpu.async_copy(pos_hbm.at[pl.ds(tb, per_w)], idx_v.at[0], s0)
        ci1 = pltpu.async_copy(pos_hbm.at[pl.ds(n_tok + tb, per_w)],
                               idx_v.at[1], s1)
        cr = pltpu.async_copy(x_hbm.at[pl.ds(tb, per_w)], rows_v, s_in)
        ci0.wait()
        ci1.wait()
        cr.wait()
        c0 = pltpu.async_copy(rows_v, out_hbm.at[idx_v.at[0]], s0)
        c1 = pltpu.async_copy(rows_v, out_hbm.at[idx_v.at[1]], s1)
        c0.wait()
        c1.wait()

    return k(xf, pos)


def _sc_gather(ys, pos):
    n_asn = pos.shape[0]
    d = ys.shape[1]
    per_w = n_asn // NW
    n_ch = per_w // CHW
    mesh = plsc.VectorSubcoreMesh(core_axis_name="c", subcore_axis_name="s")

    @functools.partial(
        pl.kernel, mesh=mesh,
        out_type=jax.ShapeDtypeStruct((n_asn, d), jnp.float32),
        scratch_types=[pltpu.VMEM((n_ch, CHW), jnp.int32),
                       pltpu.VMEM((CHW, d), jnp.float32),
                       pltpu.VMEM((CHW, d), jnp.float32),
                       pltpu.SemaphoreType.DMA,
                       pltpu.SemaphoreType.DMA],
    )
    def k(ys_hbm, pos_hbm, out_hbm, idx_v, buf_a, buf_b, sem_a, sem_b):
        wid = lax.axis_index("s") * 2 + lax.axis_index("c")
        base = wid * per_w
        for c in range(n_ch):
            pltpu.sync_copy(pos_hbm.at[pl.ds(base + c * CHW, CHW)],
                            idx_v.at[c])
        bufs = (buf_a, buf_b)
        sems = (sem_a, sem_b)
        cps = [pltpu.async_copy(ys_hbm.at[idx_v.at[0]], buf_a, sem_a)]
        for c in range(n_ch):
            cps[c].wait()
            if c + 1 < n_ch:
                cps.append(pltpu.async_copy(ys_hbm.at[idx_v.at[c + 1]],
                                            bufs[(c + 1) % 2],
                                            sems[(c + 1) % 2]))
            pltpu.sync_copy(bufs[c % 2],
                            out_hbm.at[pl.ds(base + c * CHW, CHW)])

    return k(ys, pos)


def _cast_kernel(wg_ref, wu_ref, wd_ref, og_ref, ou_ref, od_ref):
    og_ref[...] = wg_ref[...].astype(jnp.bfloat16)
    ou_ref[...] = wu_ref[...].astype(jnp.bfloat16)
    od_ref[...] = wd_ref[...].astype(jnp.bfloat16)


def _cast_weights(w_gate, w_up, w_down):
    return pl.pallas_call(
        _cast_kernel,
        grid=(E,),
        in_specs=[pl.BlockSpec((1, I, H), lambda e: (e, 0, 0)),
                  pl.BlockSpec((1, I, H), lambda e: (e, 0, 0)),
                  pl.BlockSpec((1, H, I), lambda e: (e, 0, 0))],
        out_specs=[pl.BlockSpec((1, I, H), lambda e: (e, 0, 0)),
                   pl.BlockSpec((1, I, H), lambda e: (e, 0, 0)),
                   pl.BlockSpec((1, H, I), lambda e: (e, 0, 0))],
        out_shape=[jax.ShapeDtypeStruct((E, I, H), jnp.bfloat16),
                   jax.ShapeDtypeStruct((E, I, H), jnp.bfloat16),
                   jax.ShapeDtypeStruct((E, H, I), jnp.bfloat16)],
    )(w_gate, w_up, w_down)


def _group_mm_kernel(be_ref, xs_ref, wg_ref, wu_ref, wd_ref, out_ref, *, nbe):
    b = pl.program_id(0)
    bt = be_ref[nbe - 1]

    @pl.when(b < bt)
    def _():
        xb = xs_ref[...].astype(jnp.bfloat16)
        g = lax.dot_general(xb, wg_ref[0], (((1,), (1,)), ((), ())),
                            preferred_element_type=jnp.float32)
        u = lax.dot_general(xb, wu_ref[0], (((1,), (1,)), ((), ())),
                            preferred_element_type=jnp.float32)
        h = (jax.nn.sigmoid(g) * u).astype(jnp.bfloat16)
        de = lax.dot_general(h, wd_ref[0], (((1,), (1,)), ((), ())),
                             preferred_element_type=jnp.float32)
        out_ref[...] = de.astype(jnp.bfloat16).astype(jnp.float32)


def _combine_kernel(acc_ref, y0_ref, y1_ref, w01_ref, out_ref):
    w01 = w01_ref[...].astype(jnp.bfloat16)
    c0 = (w01[:, 0:1] * y0_ref[...].astype(jnp.bfloat16)).astype(jnp.float32)
    c1 = (w01[:, 1:2] * y1_ref[...].astype(jnp.bfloat16)).astype(jnp.float32)
    out_ref[...] = acc_ref[...] + c0 + c1


@jax.jit
def kernel(x, ws_gate, ws_up, ws_down, w_router, w_gate, w_up, w_down):
    B, S, Hx = x.shape
    n_tok = B * S
    n_asn = 2 * n_tok
    n_blocks = n_asn // T + E
    flat = x.reshape(n_tok, Hx)
    wsg = ws_gate.astype(jnp.bfloat16)
    wsu = ws_up.astype(jnp.bfloat16)
    wsd = ws_down.astype(jnp.bfloat16)
    wg, wu, wd = _cast_weights(w_gate, w_up, w_down)

    pos, w01, be = pl.pallas_call(
        functools.partial(_router_kernel, n_tok=n_tok, n_asn=n_asn,
                          n_blocks=n_blocks),
        grid=(1,),
        in_specs=[pl.BlockSpec((n_tok, H), lambda i: (0, 0)),
                  pl.BlockSpec((E, H), lambda i: (0, 0))],
        out_specs=[pl.BlockSpec((n_asn // 128, 128), lambda i: (0, 0)),
                   pl.BlockSpec((n_tok, 2), lambda i: (0, 0)),
                   pl.BlockSpec((n_blocks + E, 1), lambda i: (0, 0))],
        out_shape=[jax.ShapeDtypeStruct((n_asn // 128, 128), jnp.int32),
                   jax.ShapeDtypeStruct((n_tok, 2), jnp.float32),
                   jax.ShapeDtypeStruct((n_blocks + E, 1), jnp.int32)],
    )(flat, w_router)

    nblk = n_tok // BLK
    full = lambda shape: pl.BlockSpec(shape, lambda i: (0,) * len(shape))
    acc2 = pl.pallas_call(
        _shared_kernel,
        grid=(nblk,),
        in_specs=[pl.BlockSpec((BLK, H), lambda i: (i, 0)),
                  full((NS, I, H)), full((NS, I, H)), full((NS, H, I))],
        out_specs=pl.BlockSpec((BLK, H), lambda i: (i, 0)),
        out_shape=jax.ShapeDtypeStruct((n_tok, H), jnp.float32),
    )(flat, wsg, wsu, wsd)

    pos_flat = pos.reshape(n_asn)
    xs = _sc_scatter(flat, pos_flat, n_blocks * T)

    nbe = n_blocks + E

    def _beff(b, be_r):
        return jnp.minimum(b, jnp.maximum(be_r[nbe - 1] - 1, 0))

    ys = pl.pallas_call(
        functools.partial(_group_mm_kernel, nbe=nbe),
        grid_spec=pltpu.PrefetchScalarGridSpec(
            num_scalar_prefetch=1,
            grid=(n_blocks,),
            in_specs=[
                pl.BlockSpec((T, H), lambda b, be_r: (_beff(b, be_r), 0)),
                pl.BlockSpec((1, I, H),
                             lambda b, be_r: (be_r[_beff(b, be_r)], 0, 0)),
                pl.BlockSpec((1, I, H),
                             lambda b, be_r: (be_r[_beff(b, be_r)], 0, 0)),
                pl.BlockSpec((1, H, I),
                             lambda b, be_r: (be_r[_beff(b, be_r)], 0, 0)),
            ],
            out_specs=pl.BlockSpec((T, H), lambda b, be_r: (_beff(b, be_r), 0)),
        ),
        out_shape=jax.ShapeDtypeStruct((n_blocks * T, H), jnp.float32),
    )(be.reshape(nbe), xs, wg, wu, wd)

    y01 = _sc_gather(ys, pos_flat)

    kblk = n_tok // BLK
    out = pl.pallas_call(
        _combine_kernel,
        grid=(nblk,),
        in_specs=[pl.BlockSpec((BLK, H), lambda i: (i, 0)),
                  pl.BlockSpec((BLK, H), lambda i: (i, 0)),
                  pl.BlockSpec((BLK, H), lambda i: (i + kblk, 0)),
                  pl.BlockSpec((BLK, 2), lambda i: (i, 0))],
        out_specs=pl.BlockSpec((BLK, H), lambda i: (i, 0)),
        out_shape=jax.ShapeDtypeStruct((n_tok, H), jnp.float32),
    )(acc2, y01, y01, w01)
    return out.reshape(B, S, Hx)
